# Initial kernel scaffold; baseline (speedup 1.0000x reference)
import functools

import jax
import jax.numpy as jnp
from jax import lax
from jax.experimental import pallas as pl
from jax.experimental.pallas import tpu as pltpu
from jax.experimental.pallas import tpu_sc as plsc

B = 1024
S = 512
E = 128
NC = 2
NS = 16
NW = NC * NS
S_HALF = S // 2
ROWS_PER_W = B // (NW // 2)
NV = E // 16


def _sc_body(seq_hbm, seg_hbm, tok_hbm, segtab_hbm, pos_hbm, gamma_hbm,
             beta_hbm, out_hbm, posbuf, rowbuf, idx_a, idx_b, segibuf,
             segfbuf, segtab, gbuf, bbuf, sem):
    wid = lax.axis_index("s") * NC + lax.axis_index("c")
    bg = wid // 2
    sh = wid % 2

    pltpu.sync_copy(pos_hbm.at[pl.ds(sh * S_HALF, S_HALF)], posbuf)
    pltpu.sync_copy(segtab_hbm, segtab)
    pltpu.sync_copy(gamma_hbm, gbuf)
    pltpu.sync_copy(beta_hbm, bbuf)

    s0 = [segtab[0, pl.ds(16 * k, 16)] for k in range(NV)]
    s1 = [segtab[1, pl.ds(16 * k, 16)] for k in range(NV)]
    dvec = [s1[k] - s0[k] for k in range(NV)]
    gv = [gbuf[pl.ds(16 * k, 16)] for k in range(NV)]
    bv = [bbuf[pl.ds(16 * k, 16)] for k in range(NV)]

    def add_seg0(i, carry):
        for k in range(NV):
            posbuf[i, pl.ds(16 * k, 16)] = posbuf[i, pl.ds(16 * k, 16)] + s0[k]
        return carry

    lax.fori_loop(0, S_HALF, add_seg0, 0)

    def batch_body(bi, carry):
        b = bg * ROWS_PER_W + bi
        gbase = b * S + sh * S_HALF
        pltpu.sync_copy(seq_hbm.at[pl.ds(gbase, 128)], idx_a)
        pltpu.sync_copy(seq_hbm.at[pl.ds(gbase + 128, 128)], idx_b)
        pltpu.sync_copy(seg_hbm.at[pl.ds(gbase, S_HALF)], segibuf)
        cp1 = pltpu.async_copy(tok_hbm.at[idx_a], rowbuf.at[pl.ds(0, 128)], sem)
        cp2 = pltpu.async_copy(tok_hbm.at[idx_b], rowbuf.at[pl.ds(128, 128)], sem)
        cp1.wait()
        cp2.wait()

        def conv(i, c2):
            segfbuf[pl.ds(i * 16, 16)] = segibuf[pl.ds(i * 16, 16)].astype(jnp.float32)
            return c2

        lax.fori_loop(0, S_HALF // 16, conv, 0)

        def pos_body(p, c3):
            fl = segfbuf[p]
            x = []
            for k in range(NV):
                t = rowbuf[p, pl.ds(16 * k, 16)] + posbuf[p, pl.ds(16 * k, 16)]
                x.append(t + fl * dvec[k])
            sm = (x[0] + x[1]) + (x[2] + x[3]) + ((x[4] + x[5]) + (x[6] + x[7]))
            sq = (x[0] * x[0] + x[1] * x[1]) + (x[2] * x[2] + x[3] * x[3]) + \
                 (x[4] * x[4] + x[5] * x[5]) + (x[6] * x[6] + x[7] * x[7])
            s_tot = jnp.sum(sm)
            q_tot = jnp.sum(sq)
            mean = s_tot * (1.0 / E)
            var = q_tot * (1.0 / E) - mean * mean
            v16 = jnp.broadcast_to(var + 1e-5, (16,))
            bits = plsc.bitcast(v16, jnp.int32)
            y = plsc.bitcast(jnp.int32(0x5F3759DF) - (bits >> 1), jnp.float32)
            for _ in range(3):
                y = y * (1.5 - 0.5 * v16 * y * y)
            for k in range(NV):
                rowbuf[p, pl.ds(16 * k, 16)] = (x[k] - mean) * y * gv[k] + bv[k]
            return c3

        lax.fori_loop(0, S_HALF, pos_body, 0)
        pltpu.sync_copy(rowbuf, out_hbm.at[b, pl.ds(sh * S_HALF, S_HALF)])
        return carry

    lax.fori_loop(0, ROWS_PER_W, batch_body, 0)


@jax.jit
def _sc_embed_ln(seq_flat, seg_flat, tok_table, seg_table, pos_table, gamma, beta):
    f = functools.partial(
        pl.kernel,
        out_type=jax.ShapeDtypeStruct((B, S, E), jnp.float32),
        mesh=plsc.VectorSubcoreMesh(core_axis_name="c", subcore_axis_name="s"),
        scratch_types=[
            pltpu.VMEM((S_HALF, E), jnp.float32),
            pltpu.VMEM((S_HALF, E), jnp.float32),
            pltpu.VMEM((128,), jnp.int32),
            pltpu.VMEM((128,), jnp.int32),
            pltpu.VMEM((S_HALF,), jnp.int32),
            pltpu.VMEM((S_HALF,), jnp.float32),
            pltpu.VMEM((2, E), jnp.float32),
            pltpu.VMEM((E,), jnp.float32),
            pltpu.VMEM((E,), jnp.float32),
            pltpu.SemaphoreType.DMA,
        ],
    )(_sc_body)
    return f(seq_flat, seg_flat, tok_table, seg_table, pos_table, gamma, beta)


def kernel(seq, seg, tok_table, seg_table, pos_table, gamma, beta):
    seq_flat = seq.reshape(-1).astype(jnp.int32)
    seg_flat = seg.reshape(-1).astype(jnp.int32)
    return _sc_embed_ln(seq_flat, seg_flat, tok_table, seg_table,
                        pos_table, gamma, beta)


# speedup vs baseline: 6.3065x; 6.3065x over previous
import functools

import jax
import jax.numpy as jnp
from jax import lax
from jax.experimental import pallas as pl
from jax.experimental.pallas import tpu as pltpu
from jax.experimental.pallas import tpu_sc as plsc

B = 1024
S = 512
E = 128
NC = 2
NS = 16
NW = NC * NS
S_HALF = S // 2
ROWS_PER_W = B // (NW // 2)
NV = E // 16


def _sc_body(seq_hbm, seg_hbm, tok_hbm, segtab_hbm, pos_hbm, gamma_hbm,
             beta_hbm, out_hbm, posbuf, rowbuf, idx_a, idx_b, segibuf,
             segfbuf, segtab, gbuf, bbuf, sem):
    wid = lax.axis_index("s") * NC + lax.axis_index("c")
    bg = wid // 2
    sh = wid % 2

    pltpu.sync_copy(pos_hbm.at[pl.ds(sh * S_HALF, S_HALF)], posbuf)
    pltpu.sync_copy(segtab_hbm, segtab)
    pltpu.sync_copy(gamma_hbm, gbuf)
    pltpu.sync_copy(beta_hbm, bbuf)

    s0 = [segtab[0, pl.ds(16 * k, 16)] for k in range(NV)]
    s1 = [segtab[1, pl.ds(16 * k, 16)] for k in range(NV)]
    dvec = [s1[k] - s0[k] for k in range(NV)]
    gv = [gbuf[pl.ds(16 * k, 16)] for k in range(NV)]
    bv = [bbuf[pl.ds(16 * k, 16)] for k in range(NV)]

    def add_seg0(i, carry):
        for k in range(NV):
            posbuf[i, pl.ds(16 * k, 16)] = posbuf[i, pl.ds(16 * k, 16)] + s0[k]
        return carry

    lax.fori_loop(0, S_HALF, add_seg0, 0)

    def batch_body(bi, carry):
        b = bg * ROWS_PER_W + bi
        gbase = b * S + sh * S_HALF
        pltpu.sync_copy(seq_hbm.at[pl.ds(gbase, 128)], idx_a)
        pltpu.sync_copy(seq_hbm.at[pl.ds(gbase + 128, 128)], idx_b)
        pltpu.sync_copy(seg_hbm.at[pl.ds(gbase, S_HALF)], segibuf)
        cp1 = pltpu.async_copy(tok_hbm.at[idx_a], rowbuf.at[pl.ds(0, 128)], sem)
        cp2 = pltpu.async_copy(tok_hbm.at[idx_b], rowbuf.at[pl.ds(128, 128)], sem)
        cp1.wait()
        cp2.wait()

        def conv(i, c2):
            segfbuf[pl.ds(i * 16, 16)] = segibuf[pl.ds(i * 16, 16)].astype(jnp.float32)
            return c2

        lax.fori_loop(0, S_HALF // 16, conv, 0)

        def grp_body(g, c3):
            base = g * 16
            sgv = segfbuf[pl.ds(base, 16)]
            for j in range(16):
                p = base + j
                fl = sgv[j]
                x = []
                for k in range(NV):
                    t = rowbuf[p, pl.ds(16 * k, 16)] + posbuf[p, pl.ds(16 * k, 16)]
                    x.append(t + fl * dvec[k])
                sm = (x[0] + x[1]) + (x[2] + x[3]) + ((x[4] + x[5]) + (x[6] + x[7]))
                sq = (x[0] * x[0] + x[1] * x[1]) + (x[2] * x[2] + x[3] * x[3]) + \
                     (x[4] * x[4] + x[5] * x[5]) + (x[6] * x[6] + x[7] * x[7])
                s_tot = jnp.sum(sm)
                q_tot = jnp.sum(sq)
                mean = s_tot * (1.0 / E)
                var = q_tot * (1.0 / E) - mean * mean
                v16 = jnp.broadcast_to(var + 1e-5, (16,))
                bits = plsc.bitcast(v16, jnp.int32)
                y = plsc.bitcast(jnp.int32(0x5F3759DF) - (bits >> 1), jnp.float32)
                for _ in range(3):
                    y = y * (1.5 - 0.5 * v16 * y * y)
                for k in range(NV):
                    rowbuf[p, pl.ds(16 * k, 16)] = (x[k] - mean) * y * gv[k] + bv[k]
            return c3

        lax.fori_loop(0, S_HALF // 16, grp_body, 0)
        pltpu.sync_copy(rowbuf, out_hbm.at[b, pl.ds(sh * S_HALF, S_HALF)])
        return carry

    lax.fori_loop(0, ROWS_PER_W, batch_body, 0)


@jax.jit
def _sc_embed_ln(seq_flat, seg_flat, tok_table, seg_table, pos_table, gamma, beta):
    f = functools.partial(
        pl.kernel,
        out_type=jax.ShapeDtypeStruct((B, S, E), jnp.float32),
        mesh=plsc.VectorSubcoreMesh(core_axis_name="c", subcore_axis_name="s"),
        compiler_params=pltpu.CompilerParams(needs_layout_passes=False),
        scratch_types=[
            pltpu.VMEM((S_HALF, E), jnp.float32),
            pltpu.VMEM((S_HALF, E), jnp.float32),
            pltpu.VMEM((128,), jnp.int32),
            pltpu.VMEM((128,), jnp.int32),
            pltpu.VMEM((S_HALF,), jnp.int32),
            pltpu.VMEM((S_HALF,), jnp.float32),
            pltpu.VMEM((2, E), jnp.float32),
            pltpu.VMEM((E,), jnp.float32),
            pltpu.VMEM((E,), jnp.float32),
            pltpu.SemaphoreType.DMA,
        ],
    )(_sc_body)
    return f(seq_flat, seg_flat, tok_table, seg_table, pos_table, gamma, beta)


def kernel(seq, seg, tok_table, seg_table, pos_table, gamma, beta):
    seq_flat = seq.reshape(-1).astype(jnp.int32)
    seg_flat = seg.reshape(-1).astype(jnp.int32)
    return _sc_embed_ln(seq_flat, seg_flat, tok_table, seg_table,
                        pos_table, gamma, beta)
